# baseline (speedup 1.0000x reference)
import functools

import jax
import jax.numpy as jnp
from jax import lax
from jax.experimental import pallas as pl
from jax.experimental.pallas import tpu as pltpu
from jax.experimental.pallas import tpu_sc as plsc

D_MODEL = 768
D_FF = 3072
NUM_EXPERTS = 8
EPAD = 128
T_TOK = 2048
TB = 256
NJ = T_TOK // TB
NF = 2
FBLK = D_FF // NF
DUMP = NUM_EXPERTS * NJ
NC, NS, NL = 2, 16, 16
NW = NC * NS
PCH = (2 * T_TOK) // NW
TCH = T_TOK // NW


def _gating_body(x_ref, gw_ref, gb_ref, probs_ref, auxi_ref, auxf_ref, cnt_ref):
    logits = jnp.dot(x_ref[...], gw_ref[...], preferred_element_type=jnp.float32)
    logits = logits + gb_ref[...]
    m = jnp.max(logits, axis=1, keepdims=True)
    ex = jnp.exp(logits - m)
    p = ex / jnp.sum(ex, axis=1, keepdims=True)
    T = p.shape[0]
    lane = lax.broadcasted_iota(jnp.int32, (T, EPAD), 1)
    m1 = jnp.max(p, axis=1, keepdims=True)
    i1 = jnp.min(jnp.where(p == m1, lane, EPAD), axis=1, keepdims=True)
    sel1 = lane == i1
    pm = jnp.where(sel1, -1.0, p)
    m2 = jnp.max(pm, axis=1, keepdims=True)
    i2 = jnp.min(jnp.where(pm == m2, lane, EPAD), axis=1, keepdims=True)
    sel2 = lane == i2
    msel = jnp.where(sel1 | sel2, 1.0, 0.0)
    s = msel
    sh = 1
    while sh < T:
        s = s + jnp.concatenate([jnp.zeros((sh, EPAD), jnp.float32), s[:-sh]], axis=0)
        sh *= 2
    a = s - msel
    rank1 = jnp.sum(jnp.where(sel1, a, 0.0), axis=1, keepdims=True)
    rank2 = jnp.sum(jnp.where(sel2, a, 0.0), axis=1, keepdims=True)
    slot0 = i1 * T_TOK + rank1.astype(jnp.int32)
    slot1 = i2 * T_TOK + rank2.astype(jnp.int32)
    probs_ref[...] = p
    auxi_ref[...] = jnp.where(lane == 0, slot0, jnp.where(lane == 1, slot1, 0))
    auxf_ref[...] = jnp.where(lane == 0, m1, jnp.where(lane == 1, m2, 0.0))
    counts = s[T - 1 :, :].astype(jnp.int32)
    cnt_ref[...] = jnp.broadcast_to(counts, (8, EPAD))


def _ffn_body(cnt_ref, x_ref, w1_ref, b1_ref, w2_ref, b2_ref, y_ref, acc_ref):
    e = pl.program_id(0)
    f = pl.program_id(1)
    j = pl.program_id(2)
    nb = (cnt_ref[e] + TB - 1) // TB

    @pl.when(j < nb)
    def _():
        h = jnp.dot(x_ref[...], w1_ref[0], preferred_element_type=jnp.float32)
        h = jnp.maximum(h + b1_ref[0], 0.0)
        contrib = jnp.dot(h, w2_ref[0], preferred_element_type=jnp.float32)

        @pl.when(f == 0)
        def _():
            acc_ref[pl.ds(j * TB, TB), :] = contrib

        @pl.when(f > 0)
        def _():
            acc_ref[pl.ds(j * TB, TB), :] += contrib

        @pl.when(f == NF - 1)
        def _():
            y_ref[...] = acc_ref[pl.ds(j * TB, TB), :] + b2_ref[0]


@functools.cache
def _sc_kernels():
    mesh = plsc.VectorSubcoreMesh(core_axis_name="c", subcore_axis_name="s")

    @functools.partial(
        pl.kernel,
        mesh=mesh,
        out_type=jax.ShapeDtypeStruct((NUM_EXPERTS * T_TOK, D_MODEL), jnp.float32),
        scratch_types=[
            pltpu.VMEM((PCH,), jnp.int32),
            pltpu.VMEM((PCH,), jnp.int32),
            pltpu.VMEM((PCH, D_MODEL), jnp.float32),
            pltpu.SemaphoreType.DMA,
        ],
    )
    def _dispatch(x_hbm, tsrc_hbm, slots_hbm, xs_hbm, tsrc_v, slots_v, rows_v, sem):
        wid = lax.axis_index("c") * NS + lax.axis_index("s")
        base = wid * PCH
        pltpu.sync_copy(tsrc_hbm.at[pl.ds(base, PCH)], tsrc_v)
        pltpu.sync_copy(slots_hbm.at[pl.ds(base, PCH)], slots_v)
        pltpu.async_copy(x_hbm.at[tsrc_v], rows_v, sem).wait()
        pltpu.async_copy(rows_v, xs_hbm.at[slots_v], sem).wait()

    @functools.partial(
        pl.kernel,
        mesh=mesh,
        out_type=jax.ShapeDtypeStruct((T_TOK, D_MODEL), jnp.float32),
        scratch_types=[
            pltpu.VMEM((TCH,), jnp.int32),
            pltpu.VMEM((TCH,), jnp.int32),
            pltpu.VMEM((TCH,), jnp.float32),
            pltpu.VMEM((TCH,), jnp.float32),
            pltpu.VMEM((TCH, D_MODEL), jnp.float32),
            pltpu.VMEM((TCH, D_MODEL), jnp.float32),
            pltpu.SemaphoreType.DMA,
        ],
        compiler_params=pltpu.CompilerParams(needs_layout_passes=False),
    )
    def _combine(
        ys_hbm, s0_hbm, s1_hbm, p0_hbm, p1_hbm, out_hbm,
        s0_v, s1_v, p0_v, p1_v, r0_v, r1_v, sem,
    ):
        wid = lax.axis_index("c") * NS + lax.axis_index("s")
        base = wid * TCH
        pltpu.sync_copy(s0_hbm.at[pl.ds(base, TCH)], s0_v)
        pltpu.sync_copy(s1_hbm.at[pl.ds(base, TCH)], s1_v)
        pltpu.sync_copy(p0_hbm.at[pl.ds(base, TCH)], p0_v)
        pltpu.sync_copy(p1_hbm.at[pl.ds(base, TCH)], p1_v)
        pltpu.async_copy(ys_hbm.at[s0_v], r0_v, sem).wait()
        pltpu.async_copy(ys_hbm.at[s1_v], r1_v, sem).wait()
        lane = lax.iota(jnp.int32, NL)

        def body(t, carry):
            tsplat = jnp.full((NL,), t, jnp.int32)
            w0 = plsc.load_gather(p0_v, [tsplat])
            w1v = plsc.load_gather(p1_v, [tsplat])
            for j in range(D_MODEL // NL):
                col = j * NL + lane
                c0 = plsc.load_gather(r0_v, [tsplat, col])
                c1 = plsc.load_gather(r1_v, [tsplat, col])
                plsc.store_scatter(r0_v, [tsplat, col], w0 * c0 + w1v * c1)
            return carry

        lax.fori_loop(0, TCH, body, 0)
        pltpu.sync_copy(r0_v, out_hbm.at[pl.ds(base, TCH)])

    return _dispatch, _combine


def kernel(x, gate_w, gate_b, w1, b1, w2, b2):
    B, S, D = x.shape
    T = B * S
    x2 = x.reshape(T, D)
    gwp = jnp.pad(gate_w, ((0, 0), (0, EPAD - NUM_EXPERTS)))
    gbp = jnp.pad(gate_b, (0, EPAD - NUM_EXPERTS), constant_values=-1e30)
    gbp = gbp.reshape(1, EPAD)

    probs, auxi, auxf, cnt = pl.pallas_call(
        _gating_body,
        out_shape=(
            jax.ShapeDtypeStruct((T, EPAD), jnp.float32),
            jax.ShapeDtypeStruct((T, EPAD), jnp.int32),
            jax.ShapeDtypeStruct((T, EPAD), jnp.float32),
            jax.ShapeDtypeStruct((8, EPAD), jnp.int32),
        ),
        compiler_params=pltpu.CompilerParams(
            vmem_limit_bytes=100 * 1024 * 1024,
        ),
    )(x2, gwp, gbp)

    s0 = auxi[:, 0]
    s1 = auxi[:, 1]
    p0 = auxf[:, 0]
    p1 = auxf[:, 1]
    counts8 = cnt[0, :NUM_EXPERTS]
    tok = jnp.arange(T, dtype=jnp.int32)
    tsrc = jnp.concatenate([tok, tok])
    s_all = jnp.concatenate([s0, s1])

    _dispatch, _combine = _sc_kernels()
    xs = _dispatch(x2, tsrc, s_all)

    def _cj(e, j, c):
        return jnp.minimum(j, jnp.maximum((c[e] + TB - 1) // TB - 1, 0))

    grid_spec = pltpu.PrefetchScalarGridSpec(
        num_scalar_prefetch=1,
        grid=(NUM_EXPERTS, NF, NJ),
        in_specs=[
            pl.BlockSpec((TB, D_MODEL), lambda e, f, j, c: (e * NJ + _cj(e, j, c), 0)),
            pl.BlockSpec((1, D_MODEL, FBLK), lambda e, f, j, c: (e, 0, f)),
            pl.BlockSpec((1, 1, FBLK), lambda e, f, j, c: (e, 0, f)),
            pl.BlockSpec((1, FBLK, D_MODEL), lambda e, f, j, c: (e, f, 0)),
            pl.BlockSpec((1, 1, D_MODEL), lambda e, f, j, c: (e, 0, 0)),
        ],
        out_specs=pl.BlockSpec(
            (TB, D_MODEL),
            lambda e, f, j, c: (
                jnp.where(f == NF - 1, e * NJ + _cj(e, j, c), DUMP),
                0,
            ),
        ),
        scratch_shapes=[pltpu.VMEM((T_TOK, D_MODEL), jnp.float32)],
    )
    ys = pl.pallas_call(
        _ffn_body,
        grid_spec=grid_spec,
        out_shape=jax.ShapeDtypeStruct(((NUM_EXPERTS * NJ + 1) * TB, D_MODEL), jnp.float32),
        compiler_params=pltpu.CompilerParams(
            dimension_semantics=("arbitrary", "arbitrary", "arbitrary"),
            vmem_limit_bytes=100 * 1024 * 1024,
        ),
    )(counts8, xs, w1, b1[:, None, :], w2, b2[:, None, :])

    out2 = _combine(ys, s0, s1, p0, p1)

    return out2.reshape(B, S, D), probs[:, :NUM_EXPERTS].reshape(B, S, NUM_EXPERTS)


# speedup vs baseline: 1.4993x; 1.4993x over previous
import functools

import jax
import jax.numpy as jnp
from jax import lax
from jax.experimental import pallas as pl
from jax.experimental.pallas import tpu as pltpu
from jax.experimental.pallas import tpu_sc as plsc

D_MODEL = 768
D_FF = 3072
NUM_EXPERTS = 8
EPAD = 128
T_TOK = 2048
TB = 256
NJ = T_TOK // TB
NF = 2
FBLK = D_FF // NF
DUMP = NUM_EXPERTS * NJ
NC, NS, NL = 2, 16, 16
NW = NC * NS
PCH = (2 * T_TOK) // NW
TCH = T_TOK // NW


def _gating_body(x_ref, gw_ref, gb_ref, probs_ref, auxi_ref, auxf_ref, cnt_ref):
    logits = jnp.dot(x_ref[...], gw_ref[...], preferred_element_type=jnp.float32)
    logits = logits + gb_ref[...]
    m = jnp.max(logits, axis=1, keepdims=True)
    ex = jnp.exp(logits - m)
    p = ex / jnp.sum(ex, axis=1, keepdims=True)
    T = p.shape[0]
    lane = lax.broadcasted_iota(jnp.int32, (T, EPAD), 1)
    m1 = jnp.max(p, axis=1, keepdims=True)
    i1 = jnp.min(jnp.where(p == m1, lane, EPAD), axis=1, keepdims=True)
    sel1 = lane == i1
    pm = jnp.where(sel1, -1.0, p)
    m2 = jnp.max(pm, axis=1, keepdims=True)
    i2 = jnp.min(jnp.where(pm == m2, lane, EPAD), axis=1, keepdims=True)
    sel2 = lane == i2
    msel = jnp.where(sel1 | sel2, 1.0, 0.0)
    s = msel
    sh = 1
    while sh < T:
        s = s + jnp.concatenate([jnp.zeros((sh, EPAD), jnp.float32), s[:-sh]], axis=0)
        sh *= 2
    a = s - msel
    rank1 = jnp.sum(jnp.where(sel1, a, 0.0), axis=1, keepdims=True)
    rank2 = jnp.sum(jnp.where(sel2, a, 0.0), axis=1, keepdims=True)
    slot0 = i1 * T_TOK + rank1.astype(jnp.int32)
    slot1 = i2 * T_TOK + rank2.astype(jnp.int32)
    probs_ref[...] = p
    auxi_ref[...] = jnp.where(lane == 0, slot0, jnp.where(lane == 1, slot1, 0))
    auxf_ref[...] = jnp.where(lane == 0, m1, jnp.where(lane == 1, m2, 0.0))
    counts = s[T - 1 :, :].astype(jnp.int32)
    cnt_ref[...] = jnp.broadcast_to(counts, (8, EPAD))


def _ffn_body(
    cnt_ref, xs_ref, w1_ref, b1_ref, w2_ref, b2_ref, ys_ref,
    wb1, wb2, bb1, bb2, xbuf, ybuf, wsem, xsem, ysem, bsem,
):
    e = pl.program_id(0)
    slot = lax.rem(e, 2)
    nxt = lax.rem(e + 1, 2)

    def nb_of(ei):
        return (cnt_ref[ei] + TB - 1) // TB

    def w_copies(ei, s):
        return (
            pltpu.make_async_copy(w1_ref.at[ei], wb1.at[s], wsem.at[s]),
            pltpu.make_async_copy(w2_ref.at[ei], wb2.at[s], wsem.at[s]),
        )

    def x_copy(ei, s, j):
        return pltpu.make_async_copy(
            xs_ref.at[pl.ds((ei * NJ + j) * TB, TB)],
            xbuf.at[s, pl.ds(j * TB, TB)],
            xsem.at[s],
        )

    def start_exp(ei, s):
        c1, c2 = w_copies(ei, s)
        c1.start()
        c2.start()

        def xb(j, carry):
            x_copy(ei, s, j).start()
            return carry

        lax.fori_loop(0, nb_of(ei), xb, 0)

    @pl.when(e == 0)
    def _():
        pltpu.make_async_copy(b1_ref, bb1, bsem).start()
        pltpu.make_async_copy(b2_ref, bb2, bsem).start()
        start_exp(0, 0)

    @pl.when(e + 1 < NUM_EXPERTS)
    def _():
        start_exp(e + 1, nxt)

    @pl.when(e == 0)
    def _():
        pltpu.make_async_copy(b1_ref, bb1, bsem).wait()
        pltpu.make_async_copy(b2_ref, bb2, bsem).wait()

    c1, c2 = w_copies(e, slot)
    c1.wait()
    c2.wait()

    def xw(j, carry):
        x_copy(e, slot, j).wait()
        return carry

    lax.fori_loop(0, nb_of(e), xw, 0)

    def y_copy(j):
        return pltpu.make_async_copy(
            ybuf.at[pl.ds(j * TB, TB)],
            ys_ref.at[pl.ds((e * NJ + j) * TB, TB)],
            ysem,
        )

    def cbody(j, carry):
        xb = xbuf[slot, pl.ds(j * TB, TB), :]
        h = jnp.dot(xb, wb1[slot], preferred_element_type=jnp.float32)
        h = jnp.maximum(h + bb1[e], 0.0)
        yb = jnp.dot(h, wb2[slot], preferred_element_type=jnp.float32) + bb2[e]
        ybuf[pl.ds(j * TB, TB), :] = yb
        y_copy(j).start()
        return carry

    lax.fori_loop(0, nb_of(e), cbody, 0)

    def yw(j, carry):
        y_copy(j).wait()
        return carry

    lax.fori_loop(0, nb_of(e), yw, 0)


@functools.cache
def _sc_kernels():
    mesh = plsc.VectorSubcoreMesh(core_axis_name="c", subcore_axis_name="s")

    @functools.partial(
        pl.kernel,
        mesh=mesh,
        out_type=jax.ShapeDtypeStruct((NUM_EXPERTS * T_TOK, D_MODEL), jnp.float32),
        scratch_types=[
            pltpu.VMEM((PCH,), jnp.int32),
            pltpu.VMEM((PCH,), jnp.int32),
            pltpu.VMEM((PCH, D_MODEL), jnp.float32),
            pltpu.SemaphoreType.DMA,
        ],
    )
    def _dispatch(x_hbm, tsrc_hbm, slots_hbm, xs_hbm, tsrc_v, slots_v, rows_v, sem):
        wid = lax.axis_index("c") * NS + lax.axis_index("s")
        base = wid * PCH
        pltpu.sync_copy(tsrc_hbm.at[pl.ds(base, PCH)], tsrc_v)
        pltpu.sync_copy(slots_hbm.at[pl.ds(base, PCH)], slots_v)
        pltpu.async_copy(x_hbm.at[tsrc_v], rows_v, sem).wait()
        pltpu.async_copy(rows_v, xs_hbm.at[slots_v], sem).wait()

    @functools.partial(
        pl.kernel,
        mesh=mesh,
        out_type=jax.ShapeDtypeStruct((T_TOK, D_MODEL), jnp.float32),
        scratch_types=[
            pltpu.VMEM((TCH,), jnp.int32),
            pltpu.VMEM((TCH,), jnp.int32),
            pltpu.VMEM((TCH,), jnp.float32),
            pltpu.VMEM((TCH,), jnp.float32),
            pltpu.VMEM((TCH, D_MODEL), jnp.float32),
            pltpu.VMEM((TCH, D_MODEL), jnp.float32),
            pltpu.SemaphoreType.DMA,
        ],
        compiler_params=pltpu.CompilerParams(needs_layout_passes=False),
    )
    def _combine(
        ys_hbm, s0_hbm, s1_hbm, p0_hbm, p1_hbm, out_hbm,
        s0_v, s1_v, p0_v, p1_v, r0_v, r1_v, sem,
    ):
        wid = lax.axis_index("c") * NS + lax.axis_index("s")
        base = wid * TCH
        pltpu.sync_copy(s0_hbm.at[pl.ds(base, TCH)], s0_v)
        pltpu.sync_copy(s1_hbm.at[pl.ds(base, TCH)], s1_v)
        pltpu.sync_copy(p0_hbm.at[pl.ds(base, TCH)], p0_v)
        pltpu.sync_copy(p1_hbm.at[pl.ds(base, TCH)], p1_v)
        pltpu.async_copy(ys_hbm.at[s0_v], r0_v, sem).wait()
        pltpu.async_copy(ys_hbm.at[s1_v], r1_v, sem).wait()
        lane = lax.iota(jnp.int32, NL)

        def body(t, carry):
            tsplat = jnp.full((NL,), t, jnp.int32)
            w0 = plsc.load_gather(p0_v, [tsplat])
            w1v = plsc.load_gather(p1_v, [tsplat])
            for j in range(D_MODEL // NL):
                col = j * NL + lane
                c0 = plsc.load_gather(r0_v, [tsplat, col])
                c1 = plsc.load_gather(r1_v, [tsplat, col])
                plsc.store_scatter(r0_v, [tsplat, col], w0 * c0 + w1v * c1)
            return carry

        lax.fori_loop(0, TCH, body, 0)
        pltpu.sync_copy(r0_v, out_hbm.at[pl.ds(base, TCH)])

    return _dispatch, _combine


def kernel(x, gate_w, gate_b, w1, b1, w2, b2):
    B, S, D = x.shape
    T = B * S
    x2 = x.reshape(T, D)
    gwp = jnp.pad(gate_w, ((0, 0), (0, EPAD - NUM_EXPERTS)))
    gbp = jnp.pad(gate_b, (0, EPAD - NUM_EXPERTS), constant_values=-1e30)
    gbp = gbp.reshape(1, EPAD)

    probs, auxi, auxf, cnt = pl.pallas_call(
        _gating_body,
        out_shape=(
            jax.ShapeDtypeStruct((T, EPAD), jnp.float32),
            jax.ShapeDtypeStruct((T, EPAD), jnp.int32),
            jax.ShapeDtypeStruct((T, EPAD), jnp.float32),
            jax.ShapeDtypeStruct((8, EPAD), jnp.int32),
        ),
        compiler_params=pltpu.CompilerParams(
            vmem_limit_bytes=100 * 1024 * 1024,
        ),
    )(x2, gwp, gbp)

    s0 = auxi[:, 0]
    s1 = auxi[:, 1]
    p0 = auxf[:, 0]
    p1 = auxf[:, 1]
    counts8 = cnt[0, :NUM_EXPERTS]
    tok = jnp.arange(T, dtype=jnp.int32)
    tsrc = jnp.concatenate([tok, tok])
    s_all = jnp.concatenate([s0, s1])

    _dispatch, _combine = _sc_kernels()
    xs = _dispatch(x2, tsrc, s_all)

    grid_spec = pltpu.PrefetchScalarGridSpec(
        num_scalar_prefetch=1,
        grid=(NUM_EXPERTS,),
        in_specs=[pl.BlockSpec(memory_space=pl.ANY)] * 5,
        out_specs=pl.BlockSpec(memory_space=pl.ANY),
        scratch_shapes=[
            pltpu.VMEM((2, D_MODEL, D_FF), jnp.float32),
            pltpu.VMEM((2, D_FF, D_MODEL), jnp.float32),
            pltpu.VMEM((NUM_EXPERTS, 1, D_FF), jnp.float32),
            pltpu.VMEM((NUM_EXPERTS, 1, D_MODEL), jnp.float32),
            pltpu.VMEM((2, T_TOK, D_MODEL), jnp.float32),
            pltpu.VMEM((T_TOK, D_MODEL), jnp.float32),
            pltpu.SemaphoreType.DMA((2,)),
            pltpu.SemaphoreType.DMA((2,)),
            pltpu.SemaphoreType.DMA,
            pltpu.SemaphoreType.DMA,
        ],
    )
    ys = pl.pallas_call(
        _ffn_body,
        grid_spec=grid_spec,
        out_shape=jax.ShapeDtypeStruct((NUM_EXPERTS * T_TOK, D_MODEL), jnp.float32),
        compiler_params=pltpu.CompilerParams(
            dimension_semantics=("arbitrary",),
            vmem_limit_bytes=100 * 1024 * 1024,
        ),
    )(counts8, xs, w1, b1[:, None, :], w2, b2[:, None, :])

    out2 = _combine(ys, s0, s1, p0, p1)

    return out2.reshape(B, S, D), probs[:, :NUM_EXPERTS].reshape(B, S, NUM_EXPERTS)


# speedup vs baseline: 1.5126x; 1.0089x over previous
import functools

import jax
import jax.numpy as jnp
from jax import lax
from jax.experimental import pallas as pl
from jax.experimental.pallas import tpu as pltpu
from jax.experimental.pallas import tpu_sc as plsc

D_MODEL = 768
D_FF = 3072
NUM_EXPERTS = 8
T_TOK = 2048
TB = 256
NJ = T_TOK // TB
NC, NS, NL = 2, 16, 16
NW = NC * NS
PCH = (2 * T_TOK) // NW
TCH = T_TOK // NW
CC = TCH // 2


def _gating_body(x_ref, gw_ref, gb_ref, probs_ref, auxi_ref, auxf_ref, cnt_ref):
    E = NUM_EXPERTS
    logits = jnp.dot(x_ref[...], gw_ref[...], preferred_element_type=jnp.float32)
    logits = logits + gb_ref[...]
    m = jnp.max(logits, axis=1, keepdims=True)
    ex = jnp.exp(logits - m)
    p = ex / jnp.sum(ex, axis=1, keepdims=True)
    T = p.shape[0]
    lane = lax.broadcasted_iota(jnp.int32, (T, E), 1)
    m1 = jnp.max(p, axis=1, keepdims=True)
    i1 = jnp.min(jnp.where(p == m1, lane, E), axis=1, keepdims=True)
    sel1 = lane == i1
    pm = jnp.where(sel1, -1.0, p)
    m2 = jnp.max(pm, axis=1, keepdims=True)
    i2 = jnp.min(jnp.where(pm == m2, lane, E), axis=1, keepdims=True)
    sel2 = lane == i2
    msel = jnp.where(sel1 | sel2, 1.0, 0.0)
    s = msel
    sh = 1
    while sh < T:
        s = s + jnp.concatenate([jnp.zeros((sh, E), jnp.float32), s[:-sh]], axis=0)
        sh *= 2
    a = s - msel
    rank1 = jnp.sum(jnp.where(sel1, a, 0.0), axis=1, keepdims=True)
    rank2 = jnp.sum(jnp.where(sel2, a, 0.0), axis=1, keepdims=True)
    slot0 = i1 * T_TOK + rank1.astype(jnp.int32)
    slot1 = i2 * T_TOK + rank2.astype(jnp.int32)
    probs_ref[...] = p
    auxi_ref[...] = jnp.where(lane == 0, slot0, jnp.where(lane == 1, slot1, 0))
    auxf_ref[...] = jnp.where(lane == 0, m1, jnp.where(lane == 1, m2, 0.0))
    cnt_ref[...] = jnp.broadcast_to(s[T - 1 :, :].astype(jnp.int32), (8, E))


def _ffn_body(
    cnt_ref, xs_ref, w1_ref, b1_ref, w2_ref, b2_ref, ys_ref,
    wb1, wb2, bb1, bb2, xbuf, ybuf, wsem, xsem, ysem, bsem,
):
    e = pl.program_id(0)
    slot = lax.rem(e, 2)
    nxt = lax.rem(e + 1, 2)

    def nb_of(ei):
        return (cnt_ref[ei] + TB - 1) // TB

    def w_copies(ei, s):
        return (
            pltpu.make_async_copy(w1_ref.at[ei], wb1.at[s], wsem.at[s]),
            pltpu.make_async_copy(w2_ref.at[ei], wb2.at[s], wsem.at[s]),
        )

    def x_copy(ei, s, j):
        return pltpu.make_async_copy(
            xs_ref.at[pl.ds((ei * NJ + j) * TB, TB)],
            xbuf.at[s, pl.ds(j * TB, TB)],
            xsem.at[s],
        )

    def start_exp(ei, s):
        c1, c2 = w_copies(ei, s)
        c1.start()
        c2.start()

        def xb(j, carry):
            x_copy(ei, s, j).start()
            return carry

        lax.fori_loop(0, nb_of(ei), xb, 0)

    @pl.when(e == 0)
    def _():
        pltpu.make_async_copy(b1_ref, bb1, bsem).start()
        pltpu.make_async_copy(b2_ref, bb2, bsem).start()
        start_exp(0, 0)

    @pl.when(e + 1 < NUM_EXPERTS)
    def _():
        start_exp(e + 1, nxt)

    @pl.when(e == 0)
    def _():
        pltpu.make_async_copy(b1_ref, bb1, bsem).wait()
        pltpu.make_async_copy(b2_ref, bb2, bsem).wait()

    c1, c2 = w_copies(e, slot)
    c1.wait()
    c2.wait()

    def xw(j, carry):
        x_copy(e, slot, j).wait()
        return carry

    lax.fori_loop(0, nb_of(e), xw, 0)

    def y_copy(j):
        return pltpu.make_async_copy(
            ybuf.at[pl.ds(j * TB, TB)],
            ys_ref.at[pl.ds((e * NJ + j) * TB, TB)],
            ysem,
        )

    def cbody(j, carry):
        xb = xbuf[slot, pl.ds(j * TB, TB), :]
        h = jnp.dot(xb, wb1[slot], preferred_element_type=jnp.float32)
        h = jnp.maximum(h + bb1[e], 0.0)
        yb = jnp.dot(h, wb2[slot], preferred_element_type=jnp.float32) + bb2[e]
        ybuf[pl.ds(j * TB, TB), :] = yb
        y_copy(j).start()
        return carry

    lax.fori_loop(0, nb_of(e), cbody, 0)

    def yw(j, carry):
        y_copy(j).wait()
        return carry

    lax.fori_loop(0, nb_of(e), yw, 0)


@functools.cache
def _sc_kernels():
    mesh = plsc.VectorSubcoreMesh(core_axis_name="c", subcore_axis_name="s")

    @functools.partial(
        pl.kernel,
        mesh=mesh,
        out_type=jax.ShapeDtypeStruct((NUM_EXPERTS * T_TOK, D_MODEL), jnp.float32),
        scratch_types=[
            pltpu.VMEM((PCH,), jnp.int32),
            pltpu.VMEM((PCH,), jnp.int32),
            pltpu.VMEM((PCH, D_MODEL), jnp.float32),
            pltpu.SemaphoreType.DMA,
        ],
    )
    def _dispatch(x_hbm, tsrc_hbm, slots_hbm, xs_hbm, tsrc_v, slots_v, rows_v, sem):
        wid = lax.axis_index("c") * NS + lax.axis_index("s")
        base = wid * PCH
        pltpu.sync_copy(tsrc_hbm.at[pl.ds(base, PCH)], tsrc_v)
        pltpu.sync_copy(slots_hbm.at[pl.ds(base, PCH)], slots_v)
        pltpu.async_copy(x_hbm.at[tsrc_v], rows_v, sem).wait()
        pltpu.async_copy(rows_v, xs_hbm.at[slots_v], sem).wait()

    @functools.partial(
        pl.kernel,
        mesh=mesh,
        out_type=jax.ShapeDtypeStruct((T_TOK, D_MODEL), jnp.float32),
        scratch_types=[
            pltpu.VMEM((TCH,), jnp.int32),
            pltpu.VMEM((TCH,), jnp.int32),
            pltpu.VMEM((TCH,), jnp.float32),
            pltpu.VMEM((TCH,), jnp.float32),
            pltpu.VMEM((TCH, D_MODEL), jnp.float32),
            pltpu.VMEM((TCH, D_MODEL), jnp.float32),
            pltpu.SemaphoreType.DMA,
        ],
        compiler_params=pltpu.CompilerParams(needs_layout_passes=False),
    )
    def _combine(
        ys_hbm, s0_hbm, s1_hbm, p0_hbm, p1_hbm, out_hbm,
        s0_v, s1_v, p0_v, p1_v, r0_v, r1_v, sem,
    ):
        wid = lax.axis_index("c") * NS + lax.axis_index("s")
        base = wid * TCH
        pltpu.sync_copy(s0_hbm.at[pl.ds(base, TCH)], s0_v)
        pltpu.sync_copy(s1_hbm.at[pl.ds(base, TCH)], s1_v)
        pltpu.sync_copy(p0_hbm.at[pl.ds(base, TCH)], p0_v)
        pltpu.sync_copy(p1_hbm.at[pl.ds(base, TCH)], p1_v)
        pltpu.async_copy(ys_hbm.at[s0_v], r0_v, sem).wait()
        pltpu.async_copy(ys_hbm.at[s1_v], r1_v, sem).wait()
        lane = lax.iota(jnp.int32, NL)

        def body(t, carry):
            tsplat = jnp.full((NL,), t, jnp.int32)
            w0 = plsc.load_gather(p0_v, [tsplat])
            w1v = plsc.load_gather(p1_v, [tsplat])
            for j in range(D_MODEL // NL):
                col = j * NL + lane
                c0 = plsc.load_gather(r0_v, [tsplat, col])
                c1 = plsc.load_gather(r1_v, [tsplat, col])
                plsc.store_scatter(r0_v, [tsplat, col], w0 * c0 + w1v * c1)
            return carry

        lax.fori_loop(0, TCH, body, 0)
        pltpu.sync_copy(r0_v, out_hbm.at[pl.ds(base, TCH)])

    return _dispatch, _combine


def kernel(x, gate_w, gate_b, w1, b1, w2, b2):
    B, S, D = x.shape
    T = B * S
    x2 = x.reshape(T, D)
    gb2 = gate_b.reshape(1, NUM_EXPERTS)

    probs, auxi, auxf, cnt = pl.pallas_call(
        _gating_body,
        out_shape=(
            jax.ShapeDtypeStruct((T, NUM_EXPERTS), jnp.float32),
            jax.ShapeDtypeStruct((T, NUM_EXPERTS), jnp.int32),
            jax.ShapeDtypeStruct((T, NUM_EXPERTS), jnp.float32),
            jax.ShapeDtypeStruct((8, NUM_EXPERTS), jnp.int32),
        ),
        compiler_params=pltpu.CompilerParams(
            vmem_limit_bytes=100 * 1024 * 1024,
        ),
    )(x2, gate_w, gb2)

    counts8 = cnt[0]
    s0 = auxi[:, 0]
    s1 = auxi[:, 1]
    p0 = auxf[:, 0]
    p1 = auxf[:, 1]
    tok = jnp.arange(T, dtype=jnp.int32)
    tsrc = jnp.concatenate([tok, tok])
    s_all = jnp.concatenate([s0, s1])

    _dispatch, _combine = _sc_kernels()
    xs = _dispatch(x2, tsrc, s_all)

    grid_spec = pltpu.PrefetchScalarGridSpec(
        num_scalar_prefetch=1,
        grid=(NUM_EXPERTS,),
        in_specs=[pl.BlockSpec(memory_space=pl.ANY)] * 5,
        out_specs=pl.BlockSpec(memory_space=pl.ANY),
        scratch_shapes=[
            pltpu.VMEM((2, D_MODEL, D_FF), jnp.float32),
            pltpu.VMEM((2, D_FF, D_MODEL), jnp.float32),
            pltpu.VMEM((NUM_EXPERTS, 1, D_FF), jnp.float32),
            pltpu.VMEM((NUM_EXPERTS, 1, D_MODEL), jnp.float32),
            pltpu.VMEM((2, T_TOK, D_MODEL), jnp.float32),
            pltpu.VMEM((T_TOK, D_MODEL), jnp.float32),
            pltpu.SemaphoreType.DMA((2,)),
            pltpu.SemaphoreType.DMA((2,)),
            pltpu.SemaphoreType.DMA,
            pltpu.SemaphoreType.DMA,
        ],
    )
    ys = pl.pallas_call(
        _ffn_body,
        grid_spec=grid_spec,
        out_shape=jax.ShapeDtypeStruct((NUM_EXPERTS * T_TOK, D_MODEL), jnp.float32),
        compiler_params=pltpu.CompilerParams(
            dimension_semantics=("arbitrary",),
            vmem_limit_bytes=100 * 1024 * 1024,
        ),
    )(counts8, xs, w1, b1[:, None, :], w2, b2[:, None, :])

    out2 = _combine(ys, s0, s1, p0, p1)

    return out2.reshape(B, S, D), probs.reshape(B, S, NUM_EXPERTS)


# speedup vs baseline: 1.5180x; 1.0035x over previous
import functools

import jax
import jax.numpy as jnp
from jax import lax
from jax.experimental import pallas as pl
from jax.experimental.pallas import tpu as pltpu
from jax.experimental.pallas import tpu_sc as plsc

D_MODEL = 768
D_FF = 3072
NUM_EXPERTS = 8
T_TOK = 2048
TB = 256
NJ = T_TOK // TB
NC, NS, NL = 2, 16, 16
NW = NC * NS
PCH = (2 * T_TOK) // NW
TCH = T_TOK // NW
CC = TCH // 2


def _gating_body(x_ref, gw_ref, gb_ref, probs_ref, auxi_ref, auxf_ref, cnt_ref):
    E = NUM_EXPERTS
    logits = jnp.dot(x_ref[...], gw_ref[...], preferred_element_type=jnp.float32)
    logits = logits + gb_ref[...]
    m = jnp.max(logits, axis=1, keepdims=True)
    ex = jnp.exp(logits - m)
    p = ex / jnp.sum(ex, axis=1, keepdims=True)
    T = p.shape[0]
    lane = lax.broadcasted_iota(jnp.int32, (T, E), 1)
    m1 = jnp.max(p, axis=1, keepdims=True)
    i1 = jnp.min(jnp.where(p == m1, lane, E), axis=1, keepdims=True)
    sel1 = lane == i1
    pm = jnp.where(sel1, -1.0, p)
    m2 = jnp.max(pm, axis=1, keepdims=True)
    i2 = jnp.min(jnp.where(pm == m2, lane, E), axis=1, keepdims=True)
    sel2 = lane == i2
    msel = jnp.where(sel1 | sel2, 1.0, 0.0)
    s = msel
    sh = 1
    while sh < T:
        s = s + jnp.concatenate([jnp.zeros((sh, E), jnp.float32), s[:-sh]], axis=0)
        sh *= 2
    a = s - msel
    rank1 = jnp.sum(jnp.where(sel1, a, 0.0), axis=1, keepdims=True)
    rank2 = jnp.sum(jnp.where(sel2, a, 0.0), axis=1, keepdims=True)
    slot0 = i1 * T_TOK + rank1.astype(jnp.int32)
    slot1 = i2 * T_TOK + rank2.astype(jnp.int32)
    probs_ref[...] = p
    auxi_ref[...] = jnp.where(lane == 0, slot0, jnp.where(lane == 1, slot1, 0))
    auxf_ref[...] = jnp.where(lane == 0, m1, jnp.where(lane == 1, m2, 0.0))
    cnt_ref[...] = jnp.broadcast_to(s[T - 1 :, :].astype(jnp.int32), (8, E))


def _ffn_body(
    cnt_ref, xs_ref, w1_ref, b1_ref, w2_ref, b2_ref, ys_ref,
    wb1, wb2, bb1, bb2, xbuf, ybuf, wsem, xsem, ysem, bsem,
):
    e = pl.program_id(0)
    slot = lax.rem(e, 2)
    nxt = lax.rem(e + 1, 2)

    def nb_of(ei):
        return (cnt_ref[ei] + TB - 1) // TB

    def w_copies(ei, s):
        return (
            pltpu.make_async_copy(w1_ref.at[ei], wb1.at[s], wsem.at[s]),
            pltpu.make_async_copy(w2_ref.at[ei], wb2.at[s], wsem.at[s]),
        )

    def x_copy(ei, s, j):
        return pltpu.make_async_copy(
            xs_ref.at[pl.ds((ei * NJ + j) * TB, TB)],
            xbuf.at[s, pl.ds(j * TB, TB)],
            xsem.at[s],
        )

    def start_exp(ei, s):
        c1, c2 = w_copies(ei, s)
        c1.start()
        c2.start()

        def xb(j, carry):
            x_copy(ei, s, j).start()
            return carry

        lax.fori_loop(0, nb_of(ei), xb, 0)

    @pl.when(e == 0)
    def _():
        pltpu.make_async_copy(b1_ref, bb1, bsem).start()
        pltpu.make_async_copy(b2_ref, bb2, bsem).start()
        start_exp(0, 0)

    @pl.when(e + 1 < NUM_EXPERTS)
    def _():
        start_exp(e + 1, nxt)

    @pl.when(e == 0)
    def _():
        pltpu.make_async_copy(b1_ref, bb1, bsem).wait()
        pltpu.make_async_copy(b2_ref, bb2, bsem).wait()

    c1, c2 = w_copies(e, slot)
    c1.wait()
    c2.wait()

    def xw(j, carry):
        x_copy(e, slot, j).wait()
        return carry

    lax.fori_loop(0, nb_of(e), xw, 0)

    def y_copy(j):
        return pltpu.make_async_copy(
            ybuf.at[pl.ds(j * TB, TB)],
            ys_ref.at[pl.ds((e * NJ + j) * TB, TB)],
            ysem,
        )

    def cbody(j, carry):
        xb = xbuf[slot, pl.ds(j * TB, TB), :]
        h = jnp.dot(xb, wb1[slot], preferred_element_type=jnp.float32)
        h = jnp.maximum(h + bb1[e], 0.0)
        yb = jnp.dot(h, wb2[slot], preferred_element_type=jnp.float32) + bb2[e]
        ybuf[pl.ds(j * TB, TB), :] = yb
        y_copy(j).start()
        return carry

    lax.fori_loop(0, nb_of(e), cbody, 0)

    def yw(j, carry):
        y_copy(j).wait()
        return carry

    lax.fori_loop(0, nb_of(e), yw, 0)


@functools.cache
def _sc_kernels():
    mesh = plsc.VectorSubcoreMesh(core_axis_name="c", subcore_axis_name="s")

    @functools.partial(
        pl.kernel,
        mesh=mesh,
        out_type=jax.ShapeDtypeStruct((NUM_EXPERTS * T_TOK, D_MODEL), jnp.float32),
        scratch_types=[
            pltpu.VMEM((PCH,), jnp.int32),
            pltpu.VMEM((PCH,), jnp.int32),
            pltpu.VMEM((PCH, D_MODEL), jnp.float32),
            pltpu.SemaphoreType.DMA,
        ],
    )
    def _dispatch(x_hbm, tsrc_hbm, slots_hbm, xs_hbm, tsrc_v, slots_v, rows_v, sem):
        wid = lax.axis_index("c") * NS + lax.axis_index("s")
        base = wid * PCH
        pltpu.sync_copy(tsrc_hbm.at[pl.ds(base, PCH)], tsrc_v)
        pltpu.sync_copy(slots_hbm.at[pl.ds(base, PCH)], slots_v)
        pltpu.async_copy(x_hbm.at[tsrc_v], rows_v, sem).wait()
        pltpu.async_copy(rows_v, xs_hbm.at[slots_v], sem).wait()

    @functools.partial(
        pl.kernel,
        mesh=mesh,
        out_type=jax.ShapeDtypeStruct((T_TOK, D_MODEL), jnp.float32),
        scratch_types=[
            pltpu.VMEM((CC,), jnp.int32),
            pltpu.VMEM((CC,), jnp.int32),
            pltpu.VMEM((CC,), jnp.int32),
            pltpu.VMEM((CC,), jnp.int32),
            pltpu.VMEM((TCH,), jnp.float32),
            pltpu.VMEM((TCH,), jnp.float32),
            pltpu.VMEM((CC, D_MODEL), jnp.float32),
            pltpu.VMEM((CC, D_MODEL), jnp.float32),
            pltpu.VMEM((CC, D_MODEL), jnp.float32),
            pltpu.VMEM((CC, D_MODEL), jnp.float32),
            pltpu.SemaphoreType.DMA,
            pltpu.SemaphoreType.DMA,
            pltpu.SemaphoreType.DMA,
        ],
        compiler_params=pltpu.CompilerParams(needs_layout_passes=False),
    )
    def _combine(
        ys_hbm, s0_hbm, s1_hbm, p0_hbm, p1_hbm, out_hbm,
        s0a, s0b, s1a, s1b, p0_v, p1_v, r0a, r1a, r0b, r1b, sema, semb, osem,
    ):
        wid = lax.axis_index("c") * NS + lax.axis_index("s")
        base = wid * TCH
        pltpu.sync_copy(s0_hbm.at[pl.ds(base, CC)], s0a)
        pltpu.sync_copy(s0_hbm.at[pl.ds(base + CC, CC)], s0b)
        pltpu.sync_copy(s1_hbm.at[pl.ds(base, CC)], s1a)
        pltpu.sync_copy(s1_hbm.at[pl.ds(base + CC, CC)], s1b)
        pltpu.sync_copy(p0_hbm.at[pl.ds(base, TCH)], p0_v)
        pltpu.sync_copy(p1_hbm.at[pl.ds(base, TCH)], p1_v)
        ga0 = pltpu.make_async_copy(ys_hbm.at[s0a], r0a, sema)
        ga1 = pltpu.make_async_copy(ys_hbm.at[s1a], r1a, sema)
        gb0 = pltpu.make_async_copy(ys_hbm.at[s0b], r0b, semb)
        gb1 = pltpu.make_async_copy(ys_hbm.at[s1b], r1b, semb)
        ga0.start()
        ga1.start()
        gb0.start()
        gb1.start()
        lane = lax.iota(jnp.int32, NL)

        def chunk(goff, r0_v, r1_v):
            def body(i, carry):
                tsplat = jnp.full((NL,), i, jnp.int32)
                w0 = plsc.load_gather(p0_v, [goff + tsplat])
                w1v = plsc.load_gather(p1_v, [goff + tsplat])
                for j in range(D_MODEL // NL):
                    col = j * NL + lane
                    c0 = plsc.load_gather(r0_v, [tsplat, col])
                    c1 = plsc.load_gather(r1_v, [tsplat, col])
                    plsc.store_scatter(r0_v, [tsplat, col], w0 * c0 + w1v * c1)
                return carry

            lax.fori_loop(0, CC, body, 0)

        ga0.wait()
        ga1.wait()
        chunk(0, r0a, r1a)
        oa = pltpu.make_async_copy(r0a, out_hbm.at[pl.ds(base, CC)], osem)
        oa.start()
        gb0.wait()
        gb1.wait()
        chunk(CC, r0b, r1b)
        ob = pltpu.make_async_copy(r0b, out_hbm.at[pl.ds(base + CC, CC)], osem)
        ob.start()
        oa.wait()
        ob.wait()

    return _dispatch, _combine


def kernel(x, gate_w, gate_b, w1, b1, w2, b2):
    B, S, D = x.shape
    T = B * S
    x2 = x.reshape(T, D)
    gb2 = gate_b.reshape(1, NUM_EXPERTS)

    probs, auxi, auxf, cnt = pl.pallas_call(
        _gating_body,
        out_shape=(
            jax.ShapeDtypeStruct((T, NUM_EXPERTS), jnp.float32),
            jax.ShapeDtypeStruct((T, NUM_EXPERTS), jnp.int32),
            jax.ShapeDtypeStruct((T, NUM_EXPERTS), jnp.float32),
            jax.ShapeDtypeStruct((8, NUM_EXPERTS), jnp.int32),
        ),
        compiler_params=pltpu.CompilerParams(
            vmem_limit_bytes=100 * 1024 * 1024,
        ),
    )(x2, gate_w, gb2)

    counts8 = cnt[0]
    s0 = auxi[:, 0]
    s1 = auxi[:, 1]
    p0 = auxf[:, 0]
    p1 = auxf[:, 1]
    tok = jnp.arange(T, dtype=jnp.int32)
    tsrc = jnp.concatenate([tok, tok])
    s_all = jnp.concatenate([s0, s1])

    _dispatch, _combine = _sc_kernels()
    xs = _dispatch(x2, tsrc, s_all)

    grid_spec = pltpu.PrefetchScalarGridSpec(
        num_scalar_prefetch=1,
        grid=(NUM_EXPERTS,),
        in_specs=[pl.BlockSpec(memory_space=pl.ANY)] * 5,
        out_specs=pl.BlockSpec(memory_space=pl.ANY),
        scratch_shapes=[
            pltpu.VMEM((2, D_MODEL, D_FF), jnp.float32),
            pltpu.VMEM((2, D_FF, D_MODEL), jnp.float32),
            pltpu.VMEM((NUM_EXPERTS, 1, D_FF), jnp.float32),
            pltpu.VMEM((NUM_EXPERTS, 1, D_MODEL), jnp.float32),
            pltpu.VMEM((2, T_TOK, D_MODEL), jnp.float32),
            pltpu.VMEM((T_TOK, D_MODEL), jnp.float32),
            pltpu.SemaphoreType.DMA((2,)),
            pltpu.SemaphoreType.DMA((2,)),
            pltpu.SemaphoreType.DMA,
            pltpu.SemaphoreType.DMA,
        ],
    )
    ys = pl.pallas_call(
        _ffn_body,
        grid_spec=grid_spec,
        out_shape=jax.ShapeDtypeStruct((NUM_EXPERTS * T_TOK, D_MODEL), jnp.float32),
        compiler_params=pltpu.CompilerParams(
            dimension_semantics=("arbitrary",),
            vmem_limit_bytes=100 * 1024 * 1024,
        ),
    )(counts8, xs, w1, b1[:, None, :], w2, b2[:, None, :])

    out2 = _combine(ys, s0, s1, p0, p1)

    return out2.reshape(B, S, D), probs.reshape(B, S, NUM_EXPERTS)


# speedup vs baseline: 1.6872x; 1.1115x over previous
import functools

import jax
import jax.numpy as jnp
from jax import lax
from jax.experimental import pallas as pl
from jax.experimental.pallas import tpu as pltpu
from jax.experimental.pallas import tpu_sc as plsc

D_MODEL = 768
D_FF = 3072
NUM_EXPERTS = 8
T_TOK = 2048
TB = 256
NJ = T_TOK // TB
NC, NS, NL = 2, 16, 16
NW = NC * NS
PCH = (2 * T_TOK) // NW
TCH = T_TOK // NW
CC = TCH // 2


def _gating_body(x_ref, gw_ref, gb_ref, probs_ref, auxi_ref, auxf_ref, cnt_ref):
    E = NUM_EXPERTS
    logits = jnp.dot(x_ref[...], gw_ref[...], preferred_element_type=jnp.float32)
    logits = logits + gb_ref[...]
    m = jnp.max(logits, axis=1, keepdims=True)
    ex = jnp.exp(logits - m)
    p = ex / jnp.sum(ex, axis=1, keepdims=True)
    T = p.shape[0]
    lane = lax.broadcasted_iota(jnp.int32, (T, E), 1)
    m1 = jnp.max(p, axis=1, keepdims=True)
    i1 = jnp.min(jnp.where(p == m1, lane, E), axis=1, keepdims=True)
    sel1 = lane == i1
    pm = jnp.where(sel1, -1.0, p)
    m2 = jnp.max(pm, axis=1, keepdims=True)
    i2 = jnp.min(jnp.where(pm == m2, lane, E), axis=1, keepdims=True)
    sel2 = lane == i2
    msel = jnp.where(sel1 | sel2, 1.0, 0.0)
    s = msel
    sh = 1
    while sh < T:
        s = s + jnp.concatenate([jnp.zeros((sh, E), jnp.float32), s[:-sh]], axis=0)
        sh *= 2
    a = s - msel
    rank1 = jnp.sum(jnp.where(sel1, a, 0.0), axis=1, keepdims=True)
    rank2 = jnp.sum(jnp.where(sel2, a, 0.0), axis=1, keepdims=True)
    slot0 = i1 * T_TOK + rank1.astype(jnp.int32)
    slot1 = i2 * T_TOK + rank2.astype(jnp.int32)
    probs_ref[...] = p
    auxi_ref[...] = jnp.where(lane == 0, slot0, jnp.where(lane == 1, slot1, 0))
    auxf_ref[...] = jnp.where(lane == 0, m1, jnp.where(lane == 1, m2, 0.0))
    cnt_ref[...] = jnp.broadcast_to(s[T - 1 :, :].astype(jnp.int32), (8, E))


def _ffn_body(
    cnt_ref, xs_ref, w1_ref, b1_ref, w2_ref, b2_ref, ys_ref,
    wb1, wb2, bb1, bb2, xbuf, ybuf, wsem, xsem, ysem, bsem,
):
    e = pl.program_id(0)
    slot = lax.rem(e, 2)
    nxt = lax.rem(e + 1, 2)

    def nb_of(ei):
        return (cnt_ref[ei] + TB - 1) // TB

    def w_copies(ei, s):
        return (
            pltpu.make_async_copy(w1_ref.at[ei], wb1.at[s], wsem.at[s]),
            pltpu.make_async_copy(w2_ref.at[ei], wb2.at[s], wsem.at[s]),
        )

    def x_copy(ei, s, j):
        return pltpu.make_async_copy(
            xs_ref.at[pl.ds((ei * NJ + j) * TB, TB)],
            xbuf.at[s, pl.ds(j * TB, TB)],
            xsem.at[s],
        )

    def start_exp(ei, s):
        c1, c2 = w_copies(ei, s)
        c1.start()
        c2.start()

        def xb(j, carry):
            x_copy(ei, s, j).start()
            return carry

        lax.fori_loop(0, nb_of(ei), xb, 0)

    @pl.when(e == 0)
    def _():
        pltpu.make_async_copy(b1_ref, bb1, bsem).start()
        pltpu.make_async_copy(b2_ref, bb2, bsem).start()
        start_exp(0, 0)

    @pl.when(e + 1 < NUM_EXPERTS)
    def _():
        start_exp(e + 1, nxt)

    @pl.when(e == 0)
    def _():
        pltpu.make_async_copy(b1_ref, bb1, bsem).wait()
        pltpu.make_async_copy(b2_ref, bb2, bsem).wait()

    c1, c2 = w_copies(e, slot)
    c1.wait()
    c2.wait()

    def xw(j, carry):
        x_copy(e, slot, j).wait()
        return carry

    lax.fori_loop(0, nb_of(e), xw, 0)

    def y_copy(j):
        return pltpu.make_async_copy(
            ybuf.at[pl.ds(j * TB, TB)],
            ys_ref.at[pl.ds((e * NJ + j) * TB, TB)],
            ysem,
        )

    def cbody(j, carry):
        xb = xbuf[slot, pl.ds(j * TB, TB), :]
        h = jnp.dot(xb, wb1[slot], preferred_element_type=jnp.float32)
        h = jnp.maximum(h + bb1[e], 0.0)
        yb = jnp.dot(h, wb2[slot], preferred_element_type=jnp.float32) + bb2[e]
        ybuf[pl.ds(j * TB, TB), :] = yb
        y_copy(j).start()
        return carry

    lax.fori_loop(0, nb_of(e), cbody, 0)

    def yw(j, carry):
        y_copy(j).wait()
        return carry

    lax.fori_loop(0, nb_of(e), yw, 0)


@functools.cache
def _sc_kernels():
    mesh = plsc.VectorSubcoreMesh(core_axis_name="c", subcore_axis_name="s")

    @functools.partial(
        pl.kernel,
        mesh=mesh,
        out_type=jax.ShapeDtypeStruct((NUM_EXPERTS * T_TOK, D_MODEL), jnp.float32),
        scratch_types=[
            pltpu.VMEM((PCH // 2,), jnp.int32),
            pltpu.VMEM((PCH // 2,), jnp.int32),
            pltpu.VMEM((PCH // 2,), jnp.int32),
            pltpu.VMEM((PCH // 2,), jnp.int32),
            pltpu.VMEM((PCH // 2, D_MODEL), jnp.float32),
            pltpu.VMEM((PCH // 2, D_MODEL), jnp.float32),
            pltpu.SemaphoreType.DMA,
            pltpu.SemaphoreType.DMA,
            pltpu.SemaphoreType.DMA,
        ],
    )
    def _dispatch(x_hbm, tsrc_hbm, slots_hbm, xs_hbm,
                  ta, tb, sa, sb, ra, rb, sema, semb, ssem):
        wid = lax.axis_index("c") * NS + lax.axis_index("s")
        base = wid * PCH
        half = PCH // 2
        pltpu.sync_copy(tsrc_hbm.at[pl.ds(base, half)], ta)
        pltpu.sync_copy(tsrc_hbm.at[pl.ds(base + half, half)], tb)
        pltpu.sync_copy(slots_hbm.at[pl.ds(base, half)], sa)
        pltpu.sync_copy(slots_hbm.at[pl.ds(base + half, half)], sb)
        ga = pltpu.make_async_copy(x_hbm.at[ta], ra, sema)
        gb = pltpu.make_async_copy(x_hbm.at[tb], rb, semb)
        ga.start()
        gb.start()
        ga.wait()
        wa = pltpu.make_async_copy(ra, xs_hbm.at[sa], ssem)
        wa.start()
        gb.wait()
        wb = pltpu.make_async_copy(rb, xs_hbm.at[sb], ssem)
        wb.start()
        wa.wait()
        wb.wait()

    @functools.partial(
        pl.kernel,
        mesh=mesh,
        out_type=jax.ShapeDtypeStruct((T_TOK, D_MODEL), jnp.float32),
        scratch_types=[
            pltpu.VMEM((CC,), jnp.int32),
            pltpu.VMEM((CC,), jnp.int32),
            pltpu.VMEM((CC,), jnp.int32),
            pltpu.VMEM((CC,), jnp.int32),
            pltpu.VMEM((TCH,), jnp.float32),
            pltpu.VMEM((TCH,), jnp.float32),
            pltpu.VMEM((CC, D_MODEL), jnp.float32),
            pltpu.VMEM((CC, D_MODEL), jnp.float32),
            pltpu.VMEM((CC, D_MODEL), jnp.float32),
            pltpu.VMEM((CC, D_MODEL), jnp.float32),
            pltpu.SemaphoreType.DMA,
            pltpu.SemaphoreType.DMA,
            pltpu.SemaphoreType.DMA,
        ],
        compiler_params=pltpu.CompilerParams(needs_layout_passes=False),
    )
    def _combine(
        ys_hbm, s0_hbm, s1_hbm, p0_hbm, p1_hbm, out_hbm,
        s0a, s0b, s1a, s1b, p0_v, p1_v, r0a, r1a, r0b, r1b, sema, semb, osem,
    ):
        wid = lax.axis_index("c") * NS + lax.axis_index("s")
        base = wid * TCH
        pltpu.sync_copy(s0_hbm.at[pl.ds(base, CC)], s0a)
        pltpu.sync_copy(s0_hbm.at[pl.ds(base + CC, CC)], s0b)
        pltpu.sync_copy(s1_hbm.at[pl.ds(base, CC)], s1a)
        pltpu.sync_copy(s1_hbm.at[pl.ds(base + CC, CC)], s1b)
        pltpu.sync_copy(p0_hbm.at[pl.ds(base, TCH)], p0_v)
        pltpu.sync_copy(p1_hbm.at[pl.ds(base, TCH)], p1_v)
        ga0 = pltpu.make_async_copy(ys_hbm.at[s0a], r0a, sema)
        ga1 = pltpu.make_async_copy(ys_hbm.at[s1a], r1a, sema)
        gb0 = pltpu.make_async_copy(ys_hbm.at[s0b], r0b, semb)
        gb1 = pltpu.make_async_copy(ys_hbm.at[s1b], r1b, semb)
        ga0.start()
        ga1.start()
        gb0.start()
        gb1.start()
        lane = lax.iota(jnp.int32, NL)

        def chunk(goff, r0_v, r1_v):
            def body(i, carry):
                tsplat = jnp.full((NL,), i, jnp.int32)
                w0 = plsc.load_gather(p0_v, [goff + tsplat])
                w1v = plsc.load_gather(p1_v, [goff + tsplat])
                for j in range(D_MODEL // NL):
                    c0 = r0_v[i, pl.ds(j * NL, NL)]
                    c1 = r1_v[i, pl.ds(j * NL, NL)]
                    r0_v[i, pl.ds(j * NL, NL)] = w0 * c0 + w1v * c1
                return carry

            lax.fori_loop(0, CC, body, 0)

        ga0.wait()
        ga1.wait()
        chunk(0, r0a, r1a)
        oa = pltpu.make_async_copy(r0a, out_hbm.at[pl.ds(base, CC)], osem)
        oa.start()
        gb0.wait()
        gb1.wait()
        chunk(CC, r0b, r1b)
        ob = pltpu.make_async_copy(r0b, out_hbm.at[pl.ds(base + CC, CC)], osem)
        ob.start()
        oa.wait()
        ob.wait()

    return _dispatch, _combine


def kernel(x, gate_w, gate_b, w1, b1, w2, b2):
    B, S, D = x.shape
    T = B * S
    x2 = x.reshape(T, D)
    gb2 = gate_b.reshape(1, NUM_EXPERTS)

    probs, auxi, auxf, cnt = pl.pallas_call(
        _gating_body,
        out_shape=(
            jax.ShapeDtypeStruct((T, NUM_EXPERTS), jnp.float32),
            jax.ShapeDtypeStruct((T, NUM_EXPERTS), jnp.int32),
            jax.ShapeDtypeStruct((T, NUM_EXPERTS), jnp.float32),
            jax.ShapeDtypeStruct((8, NUM_EXPERTS), jnp.int32),
        ),
        compiler_params=pltpu.CompilerParams(
            vmem_limit_bytes=100 * 1024 * 1024,
        ),
    )(x2, gate_w, gb2)

    counts8 = cnt[0]
    s0 = auxi[:, 0]
    s1 = auxi[:, 1]
    p0 = auxf[:, 0]
    p1 = auxf[:, 1]
    tok = jnp.arange(T, dtype=jnp.int32)
    tsrc = jnp.concatenate([tok, tok])
    s_all = jnp.concatenate([s0, s1])

    _dispatch, _combine = _sc_kernels()
    xs = _dispatch(x2, tsrc, s_all)

    grid_spec = pltpu.PrefetchScalarGridSpec(
        num_scalar_prefetch=1,
        grid=(NUM_EXPERTS,),
        in_specs=[pl.BlockSpec(memory_space=pl.ANY)] * 5,
        out_specs=pl.BlockSpec(memory_space=pl.ANY),
        scratch_shapes=[
            pltpu.VMEM((2, D_MODEL, D_FF), jnp.float32),
            pltpu.VMEM((2, D_FF, D_MODEL), jnp.float32),
            pltpu.VMEM((NUM_EXPERTS, 1, D_FF), jnp.float32),
            pltpu.VMEM((NUM_EXPERTS, 1, D_MODEL), jnp.float32),
            pltpu.VMEM((2, T_TOK, D_MODEL), jnp.float32),
            pltpu.VMEM((T_TOK, D_MODEL), jnp.float32),
            pltpu.SemaphoreType.DMA((2,)),
            pltpu.SemaphoreType.DMA((2,)),
            pltpu.SemaphoreType.DMA,
            pltpu.SemaphoreType.DMA,
        ],
    )
    ys = pl.pallas_call(
        _ffn_body,
        grid_spec=grid_spec,
        out_shape=jax.ShapeDtypeStruct((NUM_EXPERTS * T_TOK, D_MODEL), jnp.float32),
        compiler_params=pltpu.CompilerParams(
            dimension_semantics=("arbitrary",),
            vmem_limit_bytes=100 * 1024 * 1024,
        ),
    )(counts8, xs, w1, b1[:, None, :], w2, b2[:, None, :])

    out2 = _combine(ys, s0, s1, p0, p1)

    return out2.reshape(B, S, D), probs.reshape(B, S, NUM_EXPERTS)
